# baseline (speedup 1.0000x reference)
import jax
import jax.numpy as jnp
from jax import lax
from jax.experimental import pallas as pl
from jax.experimental.pallas import tpu as pltpu
from jax.experimental.pallas import tpu_sc as plsc

N = 10000
D = 128
E = 320000
NPAD = 10112
DUMP = 10000
K = 64
EPAD = 327680
ROWS = EPAD // K
RPT = ROWS // 16
RP2 = RPT // 2
SPLIT = 248
NEG = 0.2
EPS = 1e-16

_NC, _NS = 2, 16


def _scores_body(x_ref, al_ref, ar_ref, el_ref, er_ref):
    x = x_ref[...]
    el_ref[...] = jnp.sum(x * al_ref[...], axis=1, keepdims=True)
    er_ref[...] = jnp.sum(x * ar_ref[...], axis=1, keepdims=True)


def _scores(x, al, ar):
    el, er = pl.pallas_call(
        _scores_body,
        out_shape=[jax.ShapeDtypeStruct((N, 1), jnp.float32)] * 2,
    )(x, al, ar)
    return el.reshape(N), er.reshape(N)


def _combine_body(p_ref, o_ref):
    o_ref[...] = p_ref[0, :N, :] + p_ref[1, :N, :]


def _combine(parts):
    return pl.pallas_call(
        _combine_body,
        out_shape=jax.ShapeDtypeStruct((N, D), jnp.float32),
    )(parts)


def _edge_exp(el_v, er_v, s16, d16):
    e16 = plsc.load_gather(el_v, [s16]) + plsc.load_gather(er_v, [d16])
    e16 = jnp.where(e16 >= 0.0, e16, NEG * e16)
    return jnp.exp(e16)


def _sc_body(x_hbm, el_hbm, er_hbm, src_hbm, dst_hbm, parts_hbm,
             el_v, er_v, den_v, xin0, xin1,
             sa0, sa1, da0, da1, ea0, ea1, dx0, dx1, eb,
             i0, i1, g0, g1, s0, s1, den_sh, out_sh):
    c = lax.axis_index("c")
    s = lax.axis_index("s")
    zeros16 = jnp.zeros((16,), jnp.float32)
    xin = (xin0, xin1)
    sa = (sa0, sa1)
    da = (da0, da1)
    ea = (ea0, ea1)
    dxa = (dx0, dx1)
    isem = (i0, i1)
    gsem = (g0, g1)
    ssem = (s0, s1)

    def issue_idx(gr, b):
        pltpu.async_copy(src_hbm.at[gr], sa[b], isem[b])
        pltpu.async_copy(dst_hbm.at[gr], da[b], isem[b])

    def wait_idx(b):
        pltpu.make_async_copy(src_hbm.at[0], sa[b], isem[b]).wait()
        pltpu.make_async_copy(dst_hbm.at[0], da[b], isem[b]).wait()

    def issue_gather(b):
        pltpu.async_copy(x_hbm.at[sa[b]], xin[b], gsem[b])

    def wait_gather(b):
        pltpu.make_async_copy(x_hbm.at[pl.ds(0, K)], xin[b], gsem[b]).wait()

    def wait_exp_scatter(b):
        pltpu.make_async_copy(ea[b], den_sh.at[pl.ds(0, K)],
                              ssem[b]).wait()

    def wait_out_scatter(b):
        pltpu.make_async_copy(xin[b], out_sh.at[pl.ds(0, K)],
                              ssem[b]).wait()

    for k in range(4):
        ea0[pl.ds(k * 16, 16)] = zeros16

    def _zx(r, carry):
        for q in range(8):
            xin0[r, pl.ds(q * 16, 16)] = zeros16
        return carry
    lax.fori_loop(0, K, _zx, 0)

    @pl.when(s < 15)
    def _zlo():
        for t in range(10):
            pltpu.sync_copy(ea0, den_sh.at[pl.ds(s * 640 + t * K, K)])
        for t in range(10):
            pltpu.sync_copy(xin0, out_sh.at[pl.ds(s * 640 + t * K, K)])

    @pl.when(s == 15)
    def _zhi():
        for t in range(8):
            pltpu.sync_copy(ea0, den_sh.at[pl.ds(9600 + t * K, K)])
        for t in range(8):
            pltpu.sync_copy(xin0, out_sh.at[pl.ds(9600 + t * K, K)])

    pltpu.sync_copy(el_hbm, el_v)
    pltpu.sync_copy(er_hbm, er_v)

    plsc.subcore_barrier()

    base1 = s * RPT
    issue_idx(base1, 0)
    issue_idx(base1 + 1, 1)

    def _p1(p, carry):
        for b in range(2):
            j = 2 * p + b
            wait_idx(b)

            @pl.when(p >= 1)
            def _drain():
                wait_exp_scatter(b)

            for k in range(4):
                sl = pl.ds(k * 16, 16)
                ea[b][sl] = _edge_exp(el_v, er_v, sa[b][sl], da[b][sl])
                dxa[b][sl] = da[b][sl]
            pltpu.async_copy(ea[b], den_sh.at[dxa[b]], ssem[b], add=True)
            issue_idx(base1 + jnp.minimum(j + 2, RPT - 1), b)
        return carry
    lax.fori_loop(0, RPT // 2, _p1, 0)
    wait_idx(0)
    wait_idx(1)
    wait_exp_scatter(0)
    wait_exp_scatter(1)

    plsc.subcore_barrier()

    pltpu.sync_copy(den_sh, den_v)

    n2 = jnp.where(c == 0, SPLIT, RPT - SPLIT)
    base2 = s * RPT + jnp.where(c == 0, 0, SPLIT)

    def cj(j):
        return base2 + jnp.minimum(j, n2 - 1)

    issue_idx(cj(0), 0)
    issue_idx(cj(1), 1)
    wait_idx(0)
    issue_gather(0)

    def _p2(p, carry):
        for b in range(2):
            j = 2 * p + b
            nb = 1 - b
            wait_idx(nb)
            if b == 0:
                @pl.when(p >= 1)
                def _drain0():
                    wait_out_scatter(nb)
            else:
                wait_out_scatter(nb)
            issue_gather(nb)
            wait_gather(b)
            for k in range(4):
                sl = pl.ds(k * 16, 16)
                d16 = da[b][sl]
                x16 = _edge_exp(el_v, er_v, sa[b][sl], d16)
                den16 = plsc.load_gather(den_v, [d16]) + EPS
                eb[sl] = x16 / den16
                dxa[b][sl] = d16

            def _scale(i, carry2):
                for u in range(4):
                    e = 4 * i + u
                    e16 = jnp.broadcast_to(e, (16,)).astype(jnp.int32)
                    a16 = plsc.load_gather(eb, [e16])
                    for q in range(8):
                        qs = pl.ds(q * 16, 16)
                        xin[b][e, qs] = xin[b][e, qs] * a16
                return carry2
            lax.fori_loop(0, K // 4, _scale, 0)

            pltpu.async_copy(xin[b], out_sh.at[dxa[b]], ssem[b], add=True)
            issue_idx(cj(j + 2), b)
        return carry
    lax.fori_loop(0, n2 // 2, _p2, 0)
    wait_idx(1)
    wait_gather(0)
    wait_out_scatter(1)

    plsc.subcore_barrier()

    pltpu.sync_copy(out_sh.at[pl.ds(s * (NPAD // 16), NPAD // 16)],
                    parts_hbm.at[c, pl.ds(s * (NPAD // 16), NPAD // 16)])


def _sc_edges(x, el, er, src2d, dst2d):
    mesh = plsc.VectorSubcoreMesh(
        core_axis_name="c", subcore_axis_name="s",
        num_cores=_NC, num_subcores=_NS)
    f = pl.kernel(
        _sc_body,
        out_type=jax.ShapeDtypeStruct((_NC, NPAD, D), jnp.float32),
        mesh=mesh,
        scratch_types=[
            pltpu.VMEM((NPAD,), jnp.float32),
            pltpu.VMEM((NPAD,), jnp.float32),
            pltpu.VMEM((NPAD,), jnp.float32),
            pltpu.VMEM((K, D), jnp.float32),
            pltpu.VMEM((K, D), jnp.float32),
            pltpu.VMEM((K,), jnp.int32),
            pltpu.VMEM((K,), jnp.int32),
            pltpu.VMEM((K,), jnp.int32),
            pltpu.VMEM((K,), jnp.int32),
            pltpu.VMEM((K,), jnp.float32),
            pltpu.VMEM((K,), jnp.float32),
            pltpu.VMEM((K,), jnp.int32),
            pltpu.VMEM((K,), jnp.int32),
            pltpu.VMEM((K,), jnp.float32),
            pltpu.SemaphoreType.DMA,
            pltpu.SemaphoreType.DMA,
            pltpu.SemaphoreType.DMA,
            pltpu.SemaphoreType.DMA,
            pltpu.SemaphoreType.DMA,
            pltpu.SemaphoreType.DMA,
            pltpu.VMEM_SHARED((NPAD,), jnp.float32),
            pltpu.VMEM_SHARED((NPAD, D), jnp.float32),
        ],
        compiler_params=pltpu.CompilerParams(needs_layout_passes=False),
    )
    return f(x, el, er, src2d, dst2d)


def kernel(x, edge_index, attn_l, attn_r):
    fill0 = jnp.zeros((EPAD - E,), jnp.int32)
    filln = jnp.full((EPAD - E,), DUMP, jnp.int32)
    src2d = jnp.concatenate([edge_index[0], fill0]).reshape(ROWS, K)
    dst2d = jnp.concatenate([edge_index[1], filln]).reshape(ROWS, K)
    al = attn_l.reshape(1, D)
    ar = attn_r.reshape(1, D)
    el, er = _scores(x, al, ar)
    el_p = jnp.pad(el, (0, NPAD - N))
    er_p = jnp.pad(er, (0, NPAD - N))
    parts = _sc_edges(x, el_p, er_p, src2d, dst2d)
    out = _combine(parts)
    return out.reshape(N, 1, D)


# speedup vs baseline: 1.1077x; 1.1077x over previous
import jax
import jax.numpy as jnp
from jax import lax
from jax.experimental import pallas as pl
from jax.experimental.pallas import tpu as pltpu
from jax.experimental.pallas import tpu_sc as plsc

N = 10000
D = 128
E = 320000
NPAD = 10112
DUMP = 10000
K = 64
EPAD = 327680
ROWS = EPAD // K
RPT = ROWS // 16
RP2 = RPT // 2
SPLIT = 232
NEG = 0.2
EPS = 1e-16

_NC, _NS = 2, 16


def _scores_body(x_ref, al_ref, ar_ref, el_ref, er_ref):
    x = x_ref[...]
    el_ref[...] = jnp.sum(x * al_ref[...], axis=1, keepdims=True)
    er_ref[...] = jnp.sum(x * ar_ref[...], axis=1, keepdims=True)


def _scores(x, al, ar):
    el, er = pl.pallas_call(
        _scores_body,
        out_shape=[jax.ShapeDtypeStruct((N, 1), jnp.float32)] * 2,
    )(x, al, ar)
    return el.reshape(N), er.reshape(N)


def _combine_body(p_ref, o_ref):
    o_ref[...] = p_ref[0, :N, :] + p_ref[1, :N, :]


def _combine(parts):
    return pl.pallas_call(
        _combine_body,
        out_shape=jax.ShapeDtypeStruct((N, D), jnp.float32),
    )(parts)


def _edge_exp(el_v, er_v, s16, d16):
    e16 = plsc.load_gather(el_v, [s16]) + plsc.load_gather(er_v, [d16])
    e16 = jnp.where(e16 >= 0.0, e16, NEG * e16)
    return jnp.exp(e16)


def _sc_body(x_hbm, el_hbm, er_hbm, src_hbm, dst_hbm, parts_hbm,
             el_v, er_v, den_v, xin0, xin1,
             sa0, sa1, da0, da1, ea0, ea1, dx0, dx1, eb,
             i0, i1, g0, g1, s0, s1, den_sh, out_sh):
    c = lax.axis_index("c")
    s = lax.axis_index("s")
    zeros16 = jnp.zeros((16,), jnp.float32)
    xin = (xin0, xin1)
    sa = (sa0, sa1)
    da = (da0, da1)
    ea = (ea0, ea1)
    dxa = (dx0, dx1)
    isem = (i0, i1)
    gsem = (g0, g1)
    ssem = (s0, s1)

    def issue_idx(gr, b):
        pltpu.async_copy(src_hbm.at[gr], sa[b], isem[b])
        pltpu.async_copy(dst_hbm.at[gr], da[b], isem[b])

    def wait_idx(b):
        pltpu.make_async_copy(src_hbm.at[0], sa[b], isem[b]).wait()
        pltpu.make_async_copy(dst_hbm.at[0], da[b], isem[b]).wait()

    def issue_gather(b):
        pltpu.async_copy(x_hbm.at[sa[b]], xin[b], gsem[b])

    def wait_gather(b):
        pltpu.make_async_copy(x_hbm.at[pl.ds(0, K)], xin[b], gsem[b]).wait()

    def wait_exp_scatter(b):
        pltpu.make_async_copy(ea[b], den_sh.at[pl.ds(0, K)],
                              ssem[b]).wait()

    def wait_out_scatter(b):
        pltpu.make_async_copy(xin[b], out_sh.at[pl.ds(0, K)],
                              ssem[b]).wait()

    for k in range(4):
        ea0[pl.ds(k * 16, 16)] = zeros16

    def _zx(r, carry):
        for q in range(8):
            xin0[r, pl.ds(q * 16, 16)] = zeros16
        return carry
    lax.fori_loop(0, K, _zx, 0)

    @pl.when(s < 15)
    def _zlo():
        for t in range(10):
            pltpu.sync_copy(ea0, den_sh.at[pl.ds(s * 640 + t * K, K)])
        for t in range(10):
            pltpu.sync_copy(xin0, out_sh.at[pl.ds(s * 640 + t * K, K)])

    @pl.when(s == 15)
    def _zhi():
        for t in range(8):
            pltpu.sync_copy(ea0, den_sh.at[pl.ds(9600 + t * K, K)])
        for t in range(8):
            pltpu.sync_copy(xin0, out_sh.at[pl.ds(9600 + t * K, K)])

    pltpu.sync_copy(el_hbm, el_v)
    pltpu.sync_copy(er_hbm, er_v)

    plsc.subcore_barrier()

    base1 = s * RPT
    issue_idx(base1, 0)
    issue_idx(base1 + 1, 1)

    def _p1(p, carry):
        for b in range(2):
            j = 2 * p + b
            wait_idx(b)

            @pl.when(p >= 1)
            def _drain():
                wait_exp_scatter(b)

            for k in range(4):
                sl = pl.ds(k * 16, 16)
                ea[b][sl] = _edge_exp(el_v, er_v, sa[b][sl], da[b][sl])
                dxa[b][sl] = da[b][sl]
            pltpu.async_copy(ea[b], den_sh.at[dxa[b]], ssem[b], add=True)
            issue_idx(base1 + jnp.minimum(j + 2, RPT - 1), b)
        return carry
    lax.fori_loop(0, RPT // 2, _p1, 0)
    wait_idx(0)
    wait_idx(1)
    wait_exp_scatter(0)
    wait_exp_scatter(1)

    plsc.subcore_barrier()

    pltpu.sync_copy(den_sh, den_v)

    n2 = jnp.where(c == 0, SPLIT, RPT - SPLIT)
    base2 = s * RPT + jnp.where(c == 0, 0, SPLIT)

    def cj(j):
        return base2 + jnp.minimum(j, n2 - 1)

    issue_idx(cj(0), 0)
    issue_idx(cj(1), 1)
    wait_idx(0)
    issue_gather(0)

    def _p2(p, carry):
        for b in range(2):
            j = 2 * p + b
            nb = 1 - b
            wait_idx(nb)
            if b == 0:
                @pl.when(p >= 1)
                def _drain0():
                    wait_out_scatter(nb)
            else:
                wait_out_scatter(nb)
            issue_gather(nb)
            wait_gather(b)
            for k in range(4):
                sl = pl.ds(k * 16, 16)
                d16 = da[b][sl]
                x16 = _edge_exp(el_v, er_v, sa[b][sl], d16)
                den16 = plsc.load_gather(den_v, [d16]) + EPS
                eb[sl] = x16 / den16
                dxa[b][sl] = d16

            def _scale(i, carry2):
                for u in range(4):
                    e = 4 * i + u
                    e16 = jnp.broadcast_to(e, (16,)).astype(jnp.int32)
                    a16 = plsc.load_gather(eb, [e16])
                    for q in range(8):
                        qs = pl.ds(q * 16, 16)
                        xin[b][e, qs] = xin[b][e, qs] * a16
                return carry2
            lax.fori_loop(0, K // 4, _scale, 0)

            pltpu.async_copy(xin[b], out_sh.at[dxa[b]], ssem[b], add=True)
            issue_idx(cj(j + 2), b)
        return carry
    lax.fori_loop(0, n2 // 2, _p2, 0)
    wait_idx(1)
    wait_gather(0)
    wait_out_scatter(1)

    plsc.subcore_barrier()

    pltpu.sync_copy(out_sh.at[pl.ds(s * (NPAD // 16), NPAD // 16)],
                    parts_hbm.at[c, pl.ds(s * (NPAD // 16), NPAD // 16)])


def _sc_edges(x, el, er, src2d, dst2d):
    mesh = plsc.VectorSubcoreMesh(
        core_axis_name="c", subcore_axis_name="s",
        num_cores=_NC, num_subcores=_NS)
    f = pl.kernel(
        _sc_body,
        out_type=jax.ShapeDtypeStruct((_NC, NPAD, D), jnp.float32),
        mesh=mesh,
        scratch_types=[
            pltpu.VMEM((NPAD,), jnp.float32),
            pltpu.VMEM((NPAD,), jnp.float32),
            pltpu.VMEM((NPAD,), jnp.float32),
            pltpu.VMEM((K, D), jnp.float32),
            pltpu.VMEM((K, D), jnp.float32),
            pltpu.VMEM((K,), jnp.int32),
            pltpu.VMEM((K,), jnp.int32),
            pltpu.VMEM((K,), jnp.int32),
            pltpu.VMEM((K,), jnp.int32),
            pltpu.VMEM((K,), jnp.float32),
            pltpu.VMEM((K,), jnp.float32),
            pltpu.VMEM((K,), jnp.int32),
            pltpu.VMEM((K,), jnp.int32),
            pltpu.VMEM((K,), jnp.float32),
            pltpu.SemaphoreType.DMA,
            pltpu.SemaphoreType.DMA,
            pltpu.SemaphoreType.DMA,
            pltpu.SemaphoreType.DMA,
            pltpu.SemaphoreType.DMA,
            pltpu.SemaphoreType.DMA,
            pltpu.VMEM_SHARED((NPAD,), jnp.float32),
            pltpu.VMEM_SHARED((NPAD, D), jnp.float32),
        ],
        compiler_params=pltpu.CompilerParams(needs_layout_passes=False),
    )
    return f(x, el, er, src2d, dst2d)


def kernel(x, edge_index, attn_l, attn_r):
    fill0 = jnp.zeros((EPAD - E,), jnp.int32)
    filln = jnp.full((EPAD - E,), DUMP, jnp.int32)
    src2d = jnp.concatenate([edge_index[0], fill0]).reshape(ROWS, K)
    dst2d = jnp.concatenate([edge_index[1], filln]).reshape(ROWS, K)
    al = attn_l.reshape(1, D)
    ar = attn_r.reshape(1, D)
    el, er = _scores(x, al, ar)
    el_p = jnp.pad(el, (0, NPAD - N))
    er_p = jnp.pad(er, (0, NPAD - N))
    parts = _sc_edges(x, el_p, er_p, src2d, dst2d)
    out = _combine(parts)
    return out.reshape(N, 1, D)
